# Initial kernel scaffold; baseline (speedup 1.0000x reference)
import jax
import jax.numpy as jnp
from jax.experimental import pallas as pl


def kernel(states, table):
    raise NotImplementedError("write your pallas kernel here")


# speedup vs baseline: 1.1033x; 1.1033x over previous
import functools

import jax
import jax.numpy as jnp
from jax import lax
from jax.experimental import pallas as pl
from jax.experimental.pallas import tpu as pltpu
from jax.experimental.pallas import tpu_sc as plsc

DIM = 32
B_TOTAL = 16384 * 50
NC, NS = 2, 16
NW = NC * NS
B_PER_W = B_TOTAL // NW
CHUNK = 1600
N_CHUNKS = B_PER_W // CHUNK


@functools.partial(
    pl.kernel,
    out_type=jax.ShapeDtypeStruct((B_TOTAL, DIM), jnp.float32),
    mesh=plsc.VectorSubcoreMesh(core_axis_name="c", subcore_axis_name="s"),
    scratch_types=[
        pltpu.VMEM((CHUNK,), jnp.int32),
        pltpu.VMEM((CHUNK, DIM), jnp.float32),
        pltpu.SemaphoreType.DMA,
    ],
    compiler_params=pltpu.CompilerParams(use_tc_tiling_on_sc=False),
)
def _gather_kernel(idx_hbm, table_hbm, out_hbm, idx_v, rows_v, sem):
    wid = lax.axis_index("s") * NC + lax.axis_index("c")
    base = wid * B_PER_W

    def body(i, carry):
        off = base + i * CHUNK
        pltpu.sync_copy(idx_hbm.at[pl.ds(off, CHUNK)], idx_v)
        pltpu.async_copy(table_hbm.at[idx_v], rows_v, sem).wait()
        pltpu.sync_copy(rows_v, out_hbm.at[pl.ds(off, CHUNK)])
        return carry

    lax.fori_loop(0, N_CHUNKS, body, 0)


def kernel(states, table):
    flat = states.reshape(-1)
    out = _gather_kernel(flat, table)
    return out.reshape(states.shape[0], states.shape[1], DIM)


# speedup vs baseline: 1.1098x; 1.0058x over previous
import functools

import jax
import jax.numpy as jnp
from jax import lax
from jax.experimental import pallas as pl
from jax.experimental.pallas import tpu as pltpu
from jax.experimental.pallas import tpu_sc as plsc

DIM = 32
B_TOTAL = 16384 * 50
NC, NS = 2, 16
NW = NC * NS
B_PER_W = B_TOTAL // NW
NBUF = 4
CHUNK = 800
N_CHUNKS = B_PER_W // CHUNK
N_OUTER = N_CHUNKS // NBUF


@functools.partial(
    pl.kernel,
    out_type=jax.ShapeDtypeStruct((B_TOTAL, DIM), jnp.float32),
    mesh=plsc.VectorSubcoreMesh(core_axis_name="c", subcore_axis_name="s"),
    scratch_types=[
        pltpu.VMEM((B_PER_W,), jnp.int32),
        [pltpu.VMEM((CHUNK, DIM), jnp.float32) for _ in range(NBUF)],
        [pltpu.SemaphoreType.DMA for _ in range(NBUF)],
        [pltpu.SemaphoreType.DMA for _ in range(NBUF)],
    ],
    compiler_params=pltpu.CompilerParams(use_tc_tiling_on_sc=False),
)
def _gather_kernel(idx_hbm, table_hbm, out_hbm, idx_v, rows, gsem, wsem):
    wid = lax.axis_index("s") * NC + lax.axis_index("c")
    base = wid * B_PER_W

    pltpu.sync_copy(idx_hbm.at[pl.ds(base, B_PER_W)], idx_v)

    def start_gather(j, b):
        pltpu.async_copy(
            table_hbm.at[idx_v.at[pl.ds(j * CHUNK, CHUNK)]], rows[b], gsem[b]
        )

    for b in range(NBUF):
        start_gather(b, b)

    def outer(g, carry):
        for b in range(NBUF):
            off = base + (g * NBUF + b) * CHUNK
            pltpu.make_async_copy(
                table_hbm.at[pl.ds(0, CHUNK)], rows[b], gsem[b]
            ).wait()
            pltpu.async_copy(rows[b], out_hbm.at[pl.ds(off, CHUNK)], wsem[b])
        for b in range(NBUF):
            pltpu.make_async_copy(
                rows[b], out_hbm.at[pl.ds(base, CHUNK)], wsem[b]
            ).wait()

            @pl.when(g < N_OUTER - 1)
            def _():
                start_gather((g + 1) * NBUF + b, b)

        return carry

    lax.fori_loop(0, N_OUTER, outer, 0)


def kernel(states, table):
    flat = states.reshape(-1)
    out = _gather_kernel(flat, table)
    return out.reshape(states.shape[0], states.shape[1], DIM)


# speedup vs baseline: 1.4097x; 1.2702x over previous
import functools

import jax
import jax.numpy as jnp
from jax import lax
from jax.experimental import pallas as pl
from jax.experimental.pallas import tpu as pltpu
from jax.experimental.pallas import tpu_sc as plsc

DIM = 32
NI = 16384
NJ = 50
NJP = 56
NBLK = 250000
NC, NS = 2, 16
NW = NC * NS
IW = NI // NW
HALF = 256


@functools.partial(
    pl.kernel,
    out_type=jax.ShapeDtypeStruct((NJ, DIM, NI), jnp.float32),
    mesh=plsc.VectorSubcoreMesh(core_axis_name="c", subcore_axis_name="s"),
    scratch_types=[
        pltpu.VMEM((8, IW), jnp.int32),
        pltpu.VMEM((HALF,), jnp.int32),
        pltpu.VMEM((HALF,), jnp.int32),
        pltpu.VMEM((HALF,), jnp.int32),
        pltpu.VMEM((HALF,), jnp.int32),
        pltpu.VMEM((HALF, 128), jnp.float32),
        pltpu.VMEM((HALF, 128), jnp.float32),
        pltpu.VMEM((DIM, IW), jnp.float32),
        pltpu.SemaphoreType.DMA,
        pltpu.SemaphoreType.DMA,
        pltpu.SemaphoreType.DMA,
    ],
    compiler_params=pltpu.CompilerParams(
        use_tc_tiling_on_sc=True, needs_layout_passes=False
    ),
)
def _emb_kernel(idx_hbm, table_hbm, out_hbm, idx_blk, bidx0, bidx1,
                qoff0, qoff1, big0, big1, outblk, gsem0, gsem1, wsem):
    wid = lax.axis_index("s") * NC + lax.axis_index("c")
    i0 = pl.multiple_of(wid * IW, 128)

    bidx = (bidx0, bidx1)
    qoff = (qoff0, qoff1)
    big = (big0, big1)
    gsem = (gsem0, gsem1)

    def prep(jj, h):
        def body(k, carry):
            v = idx_blk[jj, pl.ds(h * HALF + k * 16, 16)]
            bidx[h][pl.ds(k * 16, 16)] = lax.shift_right_logical(v, 2)
            qoff[h][pl.ds(k * 16, 16)] = lax.shift_left(v & 3, 5)
            return carry
        lax.fori_loop(0, HALF // 16, body, 0)

    def extract(h):
        def body(k, carry):
            row_v = lax.iota(jnp.int32, 16) + k * 16
            colb = qoff[h][pl.ds(k * 16, 16)]
            for c in range(DIM):
                vals = plsc.load_gather(big[h], [row_v, colb + c])
                outblk[c, pl.ds(h * HALF + k * 16, 16)] = vals
            return carry
        lax.fori_loop(0, HALF // 16, body, 0)

    pltpu.async_copy(outblk, out_hbm.at[0, :, pl.ds(i0, IW)], wsem)

    def do_jb(jb, carry):
        pltpu.sync_copy(
            idx_hbm.at[pl.ds(pl.multiple_of(jb * 8, 8), 8), pl.ds(i0, IW)],
            idx_blk,
        )

        def do_jj(jj, carry2):
            j = jb * 8 + jj

            @pl.when(j < NJ)
            def _():
                prep(jj, 0)
                pltpu.async_copy(table_hbm.at[bidx[0]], big[0], gsem[0])
                prep(jj, 1)
                pltpu.async_copy(table_hbm.at[bidx[1]], big[1], gsem[1])
                pltpu.make_async_copy(
                    outblk, out_hbm.at[0, :, pl.ds(i0, IW)], wsem
                ).wait()
                pltpu.make_async_copy(
                    table_hbm.at[bidx[0]], big[0], gsem[0]
                ).wait()
                extract(0)
                pltpu.make_async_copy(
                    table_hbm.at[bidx[1]], big[1], gsem[1]
                ).wait()
                extract(1)
                pltpu.async_copy(outblk, out_hbm.at[j, :, pl.ds(i0, IW)], wsem)

            return carry2

        lax.fori_loop(0, 8, do_jj, 0)
        return carry

    lax.fori_loop(0, NJP // 8, do_jb, 0)
    pltpu.make_async_copy(outblk, out_hbm.at[0, :, pl.ds(i0, IW)], wsem).wait()


def kernel(states, table):
    statesT = jnp.swapaxes(states, 0, 1)
    statesT_p = jnp.pad(statesT, ((0, NJP - NJ), (0, 0)))
    table128 = table.reshape(NBLK, 128)
    outT = _emb_kernel(statesT_p, table128)
    return jnp.transpose(outT, (2, 0, 1))
